# Initial kernel scaffold; baseline (speedup 1.0000x reference)
import jax
import jax.numpy as jnp
from jax.experimental import pallas as pl


def kernel(x, edge_index, W1, a_src1, a_dst1, b1, W2, a_src2, a_dst2, b2):
    raise NotImplementedError("write your pallas kernel here")


# speedup vs baseline: 44.9477x; 44.9477x over previous
import functools

import jax
import jax.numpy as jnp
from jax import lax
from jax.experimental import pallas as pl
from jax.experimental.pallas import tpu as pltpu
from jax.experimental.pallas import tpu_sc as plsc

_EPS = 1e-16


def _proj_body(x_ref, wa_ref, wb_ref, oa_ref, ob_ref):
    x = x_ref[...]
    oa_ref[...] = jnp.dot(x, wa_ref[...], preferred_element_type=jnp.float32)
    ob_ref[...] = jnp.dot(x, wb_ref[...], preferred_element_type=jnp.float32)


def _proj(x, wa, wb, blk=2000):
    n, k = x.shape
    da, db = wa.shape[1], wb.shape[1]
    return pl.pallas_call(
        _proj_body,
        grid=(n // blk,),
        in_specs=[
            pl.BlockSpec((blk, k), lambda i: (i, 0)),
            pl.BlockSpec((k, da), lambda i: (0, 0)),
            pl.BlockSpec((k, db), lambda i: (0, 0)),
        ],
        out_specs=[
            pl.BlockSpec((blk, da), lambda i: (i, 0)),
            pl.BlockSpec((blk, db), lambda i: (i, 0)),
        ],
        out_shape=[
            jax.ShapeDtypeStruct((n, da), jnp.float32),
            jax.ShapeDtypeStruct((n, db), jnp.float32),
        ],
    )(x, wa, wb)


def _norm_proj_body(feat_w, a0_ref, a1_ref, s_ref, b_ref, wa_ref, wb_ref,
                    oa_ref, ob_ref):
    hs = a0_ref[...] + a1_ref[...]
    den = jnp.dot(hs, s_ref[...], preferred_element_type=jnp.float32)
    h = jnp.maximum(hs[:, :feat_w] / (den + _EPS) + b_ref[...], 0.0)
    oa_ref[...] = jnp.dot(h, wa_ref[...], preferred_element_type=jnp.float32)
    ob_ref[...] = jnp.dot(h, wb_ref[...], preferred_element_type=jnp.float32)


def _norm_proj(a0, a1, sel, b, wa, wb, blk=2000):
    n, dt = a0.shape
    feat_w = sel.shape[1]
    da, db = wa.shape[1], wb.shape[1]
    return pl.pallas_call(
        functools.partial(_norm_proj_body, feat_w),
        grid=(n // blk,),
        in_specs=[
            pl.BlockSpec((blk, dt), lambda i: (i, 0)),
            pl.BlockSpec((blk, dt), lambda i: (i, 0)),
            pl.BlockSpec((dt, feat_w), lambda i: (0, 0)),
            pl.BlockSpec((1, feat_w), lambda i: (0, 0)),
            pl.BlockSpec((feat_w, da), lambda i: (0, 0)),
            pl.BlockSpec((feat_w, db), lambda i: (0, 0)),
        ],
        out_specs=[
            pl.BlockSpec((blk, da), lambda i: (i, 0)),
            pl.BlockSpec((blk, db), lambda i: (i, 0)),
        ],
        out_shape=[
            jax.ShapeDtypeStruct((n, da), jnp.float32),
            jax.ShapeDtypeStruct((n, db), jnp.float32),
        ],
    )(a0, a1, sel, b, wa, wb)


def _final_body(feat_w, a0_ref, a1_ref, s_ref, b_ref, o_ref):
    hs = a0_ref[...] + a1_ref[...]
    den = jnp.dot(hs, s_ref[...], preferred_element_type=jnp.float32)
    o_ref[...] = hs[:, :feat_w] / (den + _EPS) + b_ref[...]


def _final(a0, a1, sel, b, blk=2000):
    n, dt = a0.shape
    feat_w = sel.shape[1]
    return pl.pallas_call(
        functools.partial(_final_body, feat_w),
        grid=(n // blk,),
        in_specs=[
            pl.BlockSpec((blk, dt), lambda i: (i, 0)),
            pl.BlockSpec((blk, dt), lambda i: (i, 0)),
            pl.BlockSpec((dt, feat_w), lambda i: (0, 0)),
            pl.BlockSpec((1, feat_w), lambda i: (0, 0)),
        ],
        out_specs=pl.BlockSpec((blk, feat_w), lambda i: (i, 0)),
        out_shape=jax.ShapeDtypeStruct((n, feat_w), jnp.float32),
    )(a0, a1, sel, b)


def _make_sc_gat(n_nodes, dt, feat_cols, head_w, n_chunks, chunk):
    n_heads = feat_cols // head_w
    vregs_per_head = head_w // 16
    mesh = plsc.VectorSubcoreMesh(core_axis_name="c", subcore_axis_name="s")
    stripe = (n_nodes // 16) & ~7
    last_stripe = n_nodes - 15 * stripe

    @functools.partial(
        pl.kernel,
        mesh=mesh,
        compiler_params=pltpu.CompilerParams(use_tc_tiling_on_sc=False),
        out_type=jax.ShapeDtypeStruct((2, n_nodes, dt), jnp.float32),
        scratch_types=[
            pltpu.VMEM((chunk,), jnp.int32),
            pltpu.VMEM((chunk,), jnp.int32),
            pltpu.VMEM((chunk, dt), jnp.float32),
            pltpu.VMEM((chunk, 16), jnp.float32),
            pltpu.VMEM_SHARED((n_nodes, dt), jnp.float32),
            pltpu.SemaphoreType.DMA,
            pltpu.SemaphoreType.DMA,
        ],
    )
    def sc_gat(t_hbm, d_hbm, src_hbm, dst_hbm, out_hbm,
               sidx, didx, trows, drows, acc, sem1, sem2):
        c = lax.axis_index("c")
        s = lax.axis_index("s")
        wid = s * 2 + c

        zero16 = jnp.zeros((16,), jnp.float32)

        def zrow(r, _):
            for k16 in range(dt // 16):
                trows[r, pl.ds(k16 * 16, 16)] = zero16
            return 0

        lax.fori_loop(0, 16, zrow, 0)
        base = s * stripe
        n16 = jnp.where(s == 15, last_stripe // 16, stripe // 16)

        def zcp(j, _):
            pltpu.sync_copy(trows.at[pl.ds(0, 16)],
                            acc.at[pl.ds(base + j * 16, 16)])
            return 0

        lax.fori_loop(0, n16, zcp, 0)
        plsc.subcore_barrier()

        def chunk_body(ch, _):
            pltpu.sync_copy(src_hbm.at[wid, ch], sidx)
            pltpu.sync_copy(dst_hbm.at[wid, ch], didx)
            cp1 = pltpu.async_copy(t_hbm.at[sidx], trows, sem1)
            cp2 = pltpu.async_copy(d_hbm.at[didx], drows, sem2)
            cp1.wait()
            cp2.wait()

            def edge_body(i, _):
                sv = trows[i, pl.ds(feat_cols, 16)]
                dv = drows[i, pl.ds(0, 16)]
                t = sv + dv
                w = jnp.exp(jnp.where(t >= 0, t, 0.2 * t))
                trows[i, pl.ds(feat_cols, 16)] = w
                for h in range(n_heads):
                    bj = lax.gather(
                        w, jnp.full((16, 1), h, jnp.int32),
                        lax.GatherDimensionNumbers(
                            offset_dims=(), collapsed_slice_dims=(0,),
                            start_index_map=(0,)),
                        slice_sizes=(1,),
                        mode=lax.GatherScatterMode.PROMISE_IN_BOUNDS)
                    for v in range(vregs_per_head):
                        col = (h * vregs_per_head + v) * 16
                        trows[i, pl.ds(col, 16)] = (
                            trows[i, pl.ds(col, 16)] * bj)
                return 0

            lax.fori_loop(0, chunk, edge_body, 0)
            pltpu.sync_copy(trows, acc.at[didx], add=True)
            return 0

        lax.fori_loop(0, n_chunks, chunk_body, 0)
        plsc.subcore_barrier()

        def ocp(j, _):
            pltpu.sync_copy(acc.at[pl.ds(base + j * 16, 16)],
                            out_hbm.at[c, pl.ds(base + j * 16, 16)])
            return 0

        lax.fori_loop(0, n16, ocp, 0)

    return sc_gat


def _head_matrix(a):
    heads, ch = a.shape
    return (jnp.eye(heads, dtype=a.dtype)[:, None, :]
            * a[:, :, None]).reshape(heads * ch, heads)


def kernel(x, edge_index, W1, a_src1, a_dst1, b1, W2, a_src2, a_dst2, b2):
    n = x.shape[0]
    e = edge_index.shape[1]
    heads, hid = a_src1.shape
    feat1 = heads * hid
    out_dim = W2.shape[1]
    dt1, dt2 = feat1 + 16, out_dim + 16

    workers = 32
    chunk = 80
    n_chunks = e // (workers * chunk)

    src_w = edge_index[0].reshape(workers, n_chunks, chunk)
    dst_w = edge_index[1].reshape(workers, n_chunks, chunk)

    f32 = jnp.float32
    z8 = jnp.zeros((x.shape[1], 16 - heads), f32)
    wa1 = jnp.concatenate([W1, W1 @ _head_matrix(a_src1), z8], axis=1)
    wd1 = jnp.concatenate([W1 @ _head_matrix(a_dst1), z8], axis=1)

    t1, d1 = _proj(x, wa1, wd1)
    acc1 = _make_sc_gat(n, dt1, feat1, hid, n_chunks, chunk)(
        t1, d1, src_w, dst_w)

    sel1 = jnp.concatenate(
        [jnp.zeros((feat1, feat1), f32),
         jnp.repeat(jnp.eye(heads, dtype=f32), hid, axis=1),
         jnp.zeros((16 - heads, feat1), f32)], axis=0)

    z15 = jnp.zeros((feat1, 15), f32)
    wa2 = jnp.concatenate([W2, W2 @ a_src2.T, z15], axis=1)
    wd2 = jnp.concatenate([W2 @ a_dst2.T, z15], axis=1)

    t2, d2 = _norm_proj(acc1[0], acc1[1], sel1, b1.reshape(1, feat1),
                        wa2, wd2)
    acc2 = _make_sc_gat(n, dt2, out_dim, out_dim, n_chunks, chunk)(
        t2, d2, src_w, dst_w)

    sel2 = jnp.concatenate(
        [jnp.zeros((out_dim, out_dim), f32),
         jnp.ones((1, out_dim), f32),
         jnp.zeros((15, out_dim), f32)], axis=0)

    return _final(acc2[0], acc2[1], sel2, b2.reshape(1, out_dim))


# speedup vs baseline: 63.8019x; 1.4195x over previous
import functools

import jax
import jax.numpy as jnp
from jax import lax
from jax.experimental import pallas as pl
from jax.experimental.pallas import tpu as pltpu
from jax.experimental.pallas import tpu_sc as plsc

_EPS = 1e-16


def _proj_body(x_ref, wa_ref, wb_ref, oa_ref, ob_ref):
    x = x_ref[...]
    oa_ref[...] = jnp.dot(x, wa_ref[...], preferred_element_type=jnp.float32)
    ob_ref[...] = jnp.dot(x, wb_ref[...], preferred_element_type=jnp.float32)


def _proj(x, wa, wb, blk=2000):
    n, k = x.shape
    da, db = wa.shape[1], wb.shape[1]
    return pl.pallas_call(
        _proj_body,
        grid=(n // blk,),
        in_specs=[
            pl.BlockSpec((blk, k), lambda i: (i, 0)),
            pl.BlockSpec((k, da), lambda i: (0, 0)),
            pl.BlockSpec((k, db), lambda i: (0, 0)),
        ],
        out_specs=[
            pl.BlockSpec((blk, da), lambda i: (i, 0)),
            pl.BlockSpec((blk, db), lambda i: (i, 0)),
        ],
        out_shape=[
            jax.ShapeDtypeStruct((n, da), jnp.float32),
            jax.ShapeDtypeStruct((n, db), jnp.float32),
        ],
    )(x, wa, wb)


def _norm_proj_body(feat_w, a0_ref, a1_ref, s_ref, b_ref, wa_ref, wb_ref,
                    oa_ref, ob_ref):
    hs = a0_ref[...] + a1_ref[...]
    den = jnp.dot(hs, s_ref[...], preferred_element_type=jnp.float32)
    h = jnp.maximum(hs[:, :feat_w] / (den + _EPS) + b_ref[...], 0.0)
    oa_ref[...] = jnp.dot(h, wa_ref[...], preferred_element_type=jnp.float32)
    ob_ref[...] = jnp.dot(h, wb_ref[...], preferred_element_type=jnp.float32)


def _norm_proj(a0, a1, sel, b, wa, wb, blk=2000):
    n, dt = a0.shape
    feat_w = sel.shape[1]
    da, db = wa.shape[1], wb.shape[1]
    return pl.pallas_call(
        functools.partial(_norm_proj_body, feat_w),
        grid=(n // blk,),
        in_specs=[
            pl.BlockSpec((blk, dt), lambda i: (i, 0)),
            pl.BlockSpec((blk, dt), lambda i: (i, 0)),
            pl.BlockSpec((dt, feat_w), lambda i: (0, 0)),
            pl.BlockSpec((1, feat_w), lambda i: (0, 0)),
            pl.BlockSpec((feat_w, da), lambda i: (0, 0)),
            pl.BlockSpec((feat_w, db), lambda i: (0, 0)),
        ],
        out_specs=[
            pl.BlockSpec((blk, da), lambda i: (i, 0)),
            pl.BlockSpec((blk, db), lambda i: (i, 0)),
        ],
        out_shape=[
            jax.ShapeDtypeStruct((n, da), jnp.float32),
            jax.ShapeDtypeStruct((n, db), jnp.float32),
        ],
    )(a0, a1, sel, b, wa, wb)


def _final_body(feat_w, a0_ref, a1_ref, s_ref, b_ref, o_ref):
    hs = a0_ref[...] + a1_ref[...]
    den = jnp.dot(hs, s_ref[...], preferred_element_type=jnp.float32)
    o_ref[...] = hs[:, :feat_w] / (den + _EPS) + b_ref[...]


def _final(a0, a1, sel, b, blk=2000):
    n, dt = a0.shape
    feat_w = sel.shape[1]
    return pl.pallas_call(
        functools.partial(_final_body, feat_w),
        grid=(n // blk,),
        in_specs=[
            pl.BlockSpec((blk, dt), lambda i: (i, 0)),
            pl.BlockSpec((blk, dt), lambda i: (i, 0)),
            pl.BlockSpec((dt, feat_w), lambda i: (0, 0)),
            pl.BlockSpec((1, feat_w), lambda i: (0, 0)),
        ],
        out_specs=pl.BlockSpec((blk, feat_w), lambda i: (i, 0)),
        out_shape=jax.ShapeDtypeStruct((n, feat_w), jnp.float32),
    )(a0, a1, sel, b)


def _make_sc_gat(n_nodes, dt, feat_cols, head_w, n_chunks, chunk):
    n_heads = feat_cols // head_w
    vregs_per_head = head_w // 16
    mesh = plsc.VectorSubcoreMesh(core_axis_name="c", subcore_axis_name="s")
    stripe = (n_nodes // 16) & ~7
    last_stripe = n_nodes - 15 * stripe

    @functools.partial(
        pl.kernel,
        mesh=mesh,
        compiler_params=pltpu.CompilerParams(use_tc_tiling_on_sc=False),
        out_type=jax.ShapeDtypeStruct((2, n_nodes, dt), jnp.float32),
        scratch_types=[
            pltpu.VMEM((3, chunk), jnp.int32),
            pltpu.VMEM((3, chunk), jnp.int32),
            pltpu.VMEM((3, chunk, dt), jnp.float32),
            pltpu.VMEM((3, chunk, 16), jnp.float32),
            pltpu.VMEM((3, chunk, dt), jnp.float32),
            pltpu.VMEM_SHARED((n_nodes, dt), jnp.float32),
            pltpu.SemaphoreType.DMA,
            pltpu.SemaphoreType.DMA,
            pltpu.SemaphoreType.DMA,
            pltpu.SemaphoreType.DMA,
            pltpu.SemaphoreType.DMA,
            pltpu.SemaphoreType.DMA,
            pltpu.SemaphoreType.DMA,
            pltpu.SemaphoreType.DMA,
            pltpu.SemaphoreType.DMA,
            pltpu.SemaphoreType.DMA,
            pltpu.SemaphoreType.DMA,
            pltpu.SemaphoreType.DMA,
        ],
    )
    def sc_gat(t_hbm, d_hbm, src_hbm, dst_hbm, out_hbm,
               sidx, didx, trows, drows, srows, acc,
               gt0, gt1, gt2, gd0, gd1, gd2,
               ss0, ss1, ss2, is0, is1, is2):
        c = lax.axis_index("c")
        s = lax.axis_index("s")
        wid = s * 2 + c
        gts = (gt0, gt1, gt2)
        gds = (gd0, gd1, gd2)
        sss = (ss0, ss1, ss2)
        iss = (is0, is1, is2)

        def idx_stage(ch, b):
            pltpu.async_copy(src_hbm.at[wid, ch], sidx.at[b], iss[b])
            pltpu.async_copy(dst_hbm.at[wid, ch], didx.at[b], iss[b])

        def idx_wait(b):
            pltpu.make_async_copy(src_hbm.at[wid, 0], sidx.at[b],
                                  iss[b]).wait()
            pltpu.make_async_copy(dst_hbm.at[wid, 0], didx.at[b],
                                  iss[b]).wait()

        def gather_start(ch, b):
            pltpu.async_copy(t_hbm.at[sidx.at[b]], trows.at[b], gts[b])
            pltpu.async_copy(d_hbm.at[didx.at[b]], drows.at[b], gds[b])

        def gather_wait(b):
            pltpu.make_async_copy(t_hbm.at[sidx.at[b]], trows.at[b],
                                  gts[b]).wait()
            pltpu.make_async_copy(d_hbm.at[didx.at[b]], drows.at[b],
                                  gds[b]).wait()

        def scatter_start(b):
            pltpu.async_copy(srows.at[b], acc.at[didx.at[b]], sss[b],
                             add=True)

        def scatter_wait(b):
            pltpu.make_async_copy(srows.at[b], acc.at[didx.at[b]],
                                  sss[b]).wait()

        idx_stage(0, 0)
        idx_stage(1, 1)

        zero16 = jnp.zeros((16,), jnp.float32)

        def zrow(r, _):
            for k16 in range(dt // 16):
                srows[0, r, pl.ds(k16 * 16, 16)] = zero16
            return 0

        lax.fori_loop(0, 16, zrow, 0)
        base = s * stripe
        n16 = jnp.where(s == 15, last_stripe // 16, stripe // 16)

        def zcp(j, _):
            pltpu.sync_copy(srows.at[0, pl.ds(0, 16)],
                            acc.at[pl.ds(base + j * 16, 16)])
            return 0

        lax.fori_loop(0, n16, zcp, 0)
        plsc.subcore_barrier()

        def compute(b):
            def edge_body(i, _):
                sv = trows[b, i, pl.ds(feat_cols, 16)]
                dv = drows[b, i, pl.ds(0, 16)]
                t = sv + dv
                w = jnp.exp(jnp.where(t >= 0, t, 0.2 * t))
                srows[b, i, pl.ds(feat_cols, 16)] = w
                for h in range(n_heads):
                    bj = lax.gather(
                        w, jnp.full((16, 1), h, jnp.int32),
                        lax.GatherDimensionNumbers(
                            offset_dims=(), collapsed_slice_dims=(0,),
                            start_index_map=(0,)),
                        slice_sizes=(1,),
                        mode=lax.GatherScatterMode.PROMISE_IN_BOUNDS)
                    for v in range(vregs_per_head):
                        col = (h * vregs_per_head + v) * 16
                        srows[b, i, pl.ds(col, 16)] = (
                            trows[b, i, pl.ds(col, 16)] * bj)
                return 0

            lax.fori_loop(0, chunk, edge_body, 0)

        idx_wait(0)
        gather_start(0, 0)

        def sub_iter(ch, b):
            bn = (b + 1) % 3
            bp = (b + 2) % 3
            gather_wait(b)

            @pl.when(ch + 1 < n_chunks)
            def _():
                idx_wait(bn)
                gather_start(ch + 1, bn)

            compute(b)

            @pl.when(ch > 0)
            def _():
                scatter_wait(bp)

            scatter_start(b)

            @pl.when(ch + 2 < n_chunks)
            def _():
                idx_stage(ch + 2, bp)

        def triple_body(t3, _):
            for b in (0, 1, 2):
                sub_iter(3 * t3 + b, b)
            return 0

        n_triples = n_chunks // 3
        lax.fori_loop(0, n_triples, triple_body, 0)
        for ch in range(3 * n_triples, n_chunks):
            b = ch % 3
            bn = (b + 1) % 3
            bp = (b + 2) % 3
            gather_wait(b)
            if ch + 1 < n_chunks:
                idx_wait(bn)
                gather_start(ch + 1, bn)
            compute(b)
            if ch > 0:
                scatter_wait(bp)
            scatter_start(b)
            if ch + 2 < n_chunks:
                idx_stage(ch + 2, bp)
        scatter_wait((n_chunks - 1) % 3)
        plsc.subcore_barrier()

        def ocp(j, _):
            pltpu.sync_copy(acc.at[pl.ds(base + j * 16, 16)],
                            out_hbm.at[c, pl.ds(base + j * 16, 16)])
            return 0

        lax.fori_loop(0, n16, ocp, 0)

    return sc_gat


def _head_matrix(a):
    heads, ch = a.shape
    return (jnp.eye(heads, dtype=a.dtype)[:, None, :]
            * a[:, :, None]).reshape(heads * ch, heads)


def kernel(x, edge_index, W1, a_src1, a_dst1, b1, W2, a_src2, a_dst2, b2):
    n = x.shape[0]
    e = edge_index.shape[1]
    heads, hid = a_src1.shape
    feat1 = heads * hid
    out_dim = W2.shape[1]
    dt1, dt2 = feat1 + 16, out_dim + 16

    workers = 32
    chunk = 40
    n_chunks = e // (workers * chunk)

    src_w = edge_index[0].reshape(workers, n_chunks, chunk)
    dst_w = edge_index[1].reshape(workers, n_chunks, chunk)

    f32 = jnp.float32
    z8 = jnp.zeros((x.shape[1], 16 - heads), f32)
    wa1 = jnp.concatenate([W1, W1 @ _head_matrix(a_src1), z8], axis=1)
    wd1 = jnp.concatenate([W1 @ _head_matrix(a_dst1), z8], axis=1)

    t1, d1 = _proj(x, wa1, wd1)
    acc1 = _make_sc_gat(n, dt1, feat1, hid, n_chunks, chunk)(
        t1, d1, src_w, dst_w)

    sel1 = jnp.concatenate(
        [jnp.zeros((feat1, feat1), f32),
         jnp.repeat(jnp.eye(heads, dtype=f32), hid, axis=1),
         jnp.zeros((16 - heads, feat1), f32)], axis=0)

    z15 = jnp.zeros((feat1, 15), f32)
    wa2 = jnp.concatenate([W2, W2 @ a_src2.T, z15], axis=1)
    wd2 = jnp.concatenate([W2 @ a_dst2.T, z15], axis=1)

    t2, d2 = _norm_proj(acc1[0], acc1[1], sel1, b1.reshape(1, feat1),
                        wa2, wd2)
    acc2 = _make_sc_gat(n, dt2, out_dim, out_dim, n_chunks, chunk)(
        t2, d2, src_w, dst_w)

    sel2 = jnp.concatenate(
        [jnp.zeros((out_dim, out_dim), f32),
         jnp.ones((1, out_dim), f32),
         jnp.zeros((15, out_dim), f32)], axis=0)

    return _final(acc2[0], acc2[1], sel2, b2.reshape(1, out_dim))


# speedup vs baseline: 81.2505x; 1.2735x over previous
import functools

import jax
import jax.numpy as jnp
from jax import lax
from jax.experimental import pallas as pl
from jax.experimental.pallas import tpu as pltpu
from jax.experimental.pallas import tpu_sc as plsc

_EPS = 1e-16


def _proj_body(x_ref, wa_ref, wb_ref, oa_ref, ob_ref):
    x = x_ref[...]
    oa_ref[...] = jnp.dot(x, wa_ref[...], preferred_element_type=jnp.float32)
    ob_ref[...] = jnp.dot(x, wb_ref[...], preferred_element_type=jnp.float32)


def _proj(x, wa, wb, blk=2000):
    n, k = x.shape
    da, db = wa.shape[1], wb.shape[1]
    return pl.pallas_call(
        _proj_body,
        grid=(n // blk,),
        in_specs=[
            pl.BlockSpec((blk, k), lambda i: (i, 0)),
            pl.BlockSpec((k, da), lambda i: (0, 0)),
            pl.BlockSpec((k, db), lambda i: (0, 0)),
        ],
        out_specs=[
            pl.BlockSpec((blk, da), lambda i: (i, 0)),
            pl.BlockSpec((blk, db), lambda i: (i, 0)),
        ],
        out_shape=[
            jax.ShapeDtypeStruct((n, da), jnp.float32),
            jax.ShapeDtypeStruct((n, db), jnp.float32),
        ],
    )(x, wa, wb)


def _norm_proj_body(feat_w, a0_ref, a1_ref, s_ref, b_ref, wa_ref, wb_ref,
                    oa_ref, ob_ref):
    hs = a0_ref[...] + a1_ref[...]
    den = jnp.dot(hs, s_ref[...], preferred_element_type=jnp.float32)
    h = jnp.maximum(hs[:, :feat_w] / (den + _EPS) + b_ref[...], 0.0)
    oa_ref[...] = jnp.dot(h, wa_ref[...], preferred_element_type=jnp.float32)
    ob_ref[...] = jnp.dot(h, wb_ref[...], preferred_element_type=jnp.float32)


def _norm_proj(a0, a1, sel, b, wa, wb, blk=2000):
    n, dt = a0.shape
    feat_w = sel.shape[1]
    da, db = wa.shape[1], wb.shape[1]
    return pl.pallas_call(
        functools.partial(_norm_proj_body, feat_w),
        grid=(n // blk,),
        in_specs=[
            pl.BlockSpec((blk, dt), lambda i: (i, 0)),
            pl.BlockSpec((blk, dt), lambda i: (i, 0)),
            pl.BlockSpec((dt, feat_w), lambda i: (0, 0)),
            pl.BlockSpec((1, feat_w), lambda i: (0, 0)),
            pl.BlockSpec((feat_w, da), lambda i: (0, 0)),
            pl.BlockSpec((feat_w, db), lambda i: (0, 0)),
        ],
        out_specs=[
            pl.BlockSpec((blk, da), lambda i: (i, 0)),
            pl.BlockSpec((blk, db), lambda i: (i, 0)),
        ],
        out_shape=[
            jax.ShapeDtypeStruct((n, da), jnp.float32),
            jax.ShapeDtypeStruct((n, db), jnp.float32),
        ],
    )(a0, a1, sel, b, wa, wb)


def _final_body(feat_w, a0_ref, a1_ref, s_ref, b_ref, o_ref):
    hs = a0_ref[...] + a1_ref[...]
    den = jnp.dot(hs, s_ref[...], preferred_element_type=jnp.float32)
    o_ref[...] = hs[:, :feat_w] / (den + _EPS) + b_ref[...]


def _final(a0, a1, sel, b, blk=2000):
    n, dt = a0.shape
    feat_w = sel.shape[1]
    return pl.pallas_call(
        functools.partial(_final_body, feat_w),
        grid=(n // blk,),
        in_specs=[
            pl.BlockSpec((blk, dt), lambda i: (i, 0)),
            pl.BlockSpec((blk, dt), lambda i: (i, 0)),
            pl.BlockSpec((dt, feat_w), lambda i: (0, 0)),
            pl.BlockSpec((1, feat_w), lambda i: (0, 0)),
        ],
        out_specs=pl.BlockSpec((blk, feat_w), lambda i: (i, 0)),
        out_shape=jax.ShapeDtypeStruct((n, feat_w), jnp.float32),
    )(a0, a1, sel, b)


def _make_sc_gat(n_nodes, dt, feat_cols, head_w, n_chunks, chunk):
    n_heads = feat_cols // head_w
    vregs_per_head = head_w // 16
    mesh = plsc.VectorSubcoreMesh(core_axis_name="c", subcore_axis_name="s")
    stripe = (n_nodes // 16) & ~7
    last_stripe = n_nodes - 15 * stripe

    @functools.partial(
        pl.kernel,
        mesh=mesh,
        compiler_params=pltpu.CompilerParams(use_tc_tiling_on_sc=False),
        out_type=jax.ShapeDtypeStruct((2, n_nodes, dt), jnp.float32),
        scratch_types=[
            pltpu.VMEM((3, chunk), jnp.int32),
            pltpu.VMEM((3, chunk), jnp.int32),
            pltpu.VMEM((3, chunk, dt), jnp.float32),
            pltpu.VMEM((3, chunk, 16), jnp.float32),
            pltpu.VMEM((3, chunk, dt), jnp.float32),
            pltpu.VMEM_SHARED((n_nodes, dt), jnp.float32),
            pltpu.SemaphoreType.DMA,
            pltpu.SemaphoreType.DMA,
            pltpu.SemaphoreType.DMA,
            pltpu.SemaphoreType.DMA,
            pltpu.SemaphoreType.DMA,
            pltpu.SemaphoreType.DMA,
            pltpu.SemaphoreType.DMA,
            pltpu.SemaphoreType.DMA,
            pltpu.SemaphoreType.DMA,
            pltpu.SemaphoreType.DMA,
            pltpu.SemaphoreType.DMA,
            pltpu.SemaphoreType.DMA,
        ],
    )
    def sc_gat(t_hbm, d_hbm, src_hbm, dst_hbm, out_hbm,
               sidx, didx, trows, drows, srows, acc,
               gt0, gt1, gt2, gd0, gd1, gd2,
               ss0, ss1, ss2, is0, is1, is2):
        c = lax.axis_index("c")
        s = lax.axis_index("s")
        wid = s * 2 + c
        gts = (gt0, gt1, gt2)
        gds = (gd0, gd1, gd2)
        sss = (ss0, ss1, ss2)
        iss = (is0, is1, is2)

        def idx_stage(ch, b):
            pltpu.async_copy(src_hbm.at[wid, ch], sidx.at[b], iss[b])
            pltpu.async_copy(dst_hbm.at[wid, ch], didx.at[b], iss[b])

        def idx_wait(b):
            pltpu.make_async_copy(src_hbm.at[wid, 0], sidx.at[b],
                                  iss[b]).wait()
            pltpu.make_async_copy(dst_hbm.at[wid, 0], didx.at[b],
                                  iss[b]).wait()

        def gather_start(ch, b):
            pltpu.async_copy(t_hbm.at[sidx.at[b]], trows.at[b], gts[b])
            pltpu.async_copy(d_hbm.at[didx.at[b]], drows.at[b], gds[b])

        def gather_wait(b):
            pltpu.make_async_copy(t_hbm.at[sidx.at[b]], trows.at[b],
                                  gts[b]).wait()
            pltpu.make_async_copy(d_hbm.at[didx.at[b]], drows.at[b],
                                  gds[b]).wait()

        def scatter_start(b):
            pltpu.async_copy(srows.at[b], acc.at[didx.at[b]], sss[b],
                             add=True)

        def scatter_wait(b):
            pltpu.make_async_copy(srows.at[b], acc.at[didx.at[b]],
                                  sss[b]).wait()

        idx_stage(0, 0)
        idx_stage(1, 1)

        zero16 = jnp.zeros((16,), jnp.float32)

        def zrow(r, _):
            for k16 in range(dt // 16):
                srows[0, r, pl.ds(k16 * 16, 16)] = zero16
            return 0

        lax.fori_loop(0, 16, zrow, 0)
        base = s * stripe
        n16 = jnp.where(s == 15, last_stripe // 16, stripe // 16)

        def zcp(j, _):
            pltpu.sync_copy(srows.at[0, pl.ds(0, 16)],
                            acc.at[pl.ds(base + j * 16, 16)])
            return 0

        lax.fori_loop(0, n16, zcp, 0)
        plsc.subcore_barrier()

        def compute(b):
            @plsc.parallel_loop(0, chunk, unroll=4)
            def edge_body(i):
                sv = trows[b, i, pl.ds(feat_cols, 16)]
                dv = drows[b, i, pl.ds(0, 16)]
                t = sv + dv
                w = jnp.exp(jnp.where(t >= 0, t, 0.2 * t))
                srows[b, i, pl.ds(feat_cols, 16)] = w
                for h in range(n_heads):
                    bj = lax.gather(
                        w, jnp.full((16, 1), h, jnp.int32),
                        lax.GatherDimensionNumbers(
                            offset_dims=(), collapsed_slice_dims=(0,),
                            start_index_map=(0,)),
                        slice_sizes=(1,),
                        mode=lax.GatherScatterMode.PROMISE_IN_BOUNDS)
                    for v in range(vregs_per_head):
                        col = (h * vregs_per_head + v) * 16
                        srows[b, i, pl.ds(col, 16)] = (
                            trows[b, i, pl.ds(col, 16)] * bj)

        idx_wait(0)
        gather_start(0, 0)

        def sub_iter(ch, b):
            bn = (b + 1) % 3
            bp = (b + 2) % 3
            gather_wait(b)

            @pl.when(ch + 1 < n_chunks)
            def _():
                idx_wait(bn)
                gather_start(ch + 1, bn)

            compute(b)

            @pl.when(ch > 0)
            def _():
                scatter_wait(bp)

            scatter_start(b)

            @pl.when(ch + 2 < n_chunks)
            def _():
                idx_stage(ch + 2, bp)

        def triple_body(t3, _):
            for b in (0, 1, 2):
                sub_iter(3 * t3 + b, b)
            return 0

        n_triples = n_chunks // 3
        lax.fori_loop(0, n_triples, triple_body, 0)
        for ch in range(3 * n_triples, n_chunks):
            b = ch % 3
            bn = (b + 1) % 3
            bp = (b + 2) % 3
            gather_wait(b)
            if ch + 1 < n_chunks:
                idx_wait(bn)
                gather_start(ch + 1, bn)
            compute(b)
            if ch > 0:
                scatter_wait(bp)
            scatter_start(b)
            if ch + 2 < n_chunks:
                idx_stage(ch + 2, bp)
        scatter_wait((n_chunks - 1) % 3)
        plsc.subcore_barrier()

        def ocp(j, _):
            pltpu.sync_copy(acc.at[pl.ds(base + j * 16, 16)],
                            out_hbm.at[c, pl.ds(base + j * 16, 16)])
            return 0

        lax.fori_loop(0, n16, ocp, 0)

    return sc_gat


def _head_matrix(a):
    heads, ch = a.shape
    return (jnp.eye(heads, dtype=a.dtype)[:, None, :]
            * a[:, :, None]).reshape(heads * ch, heads)


def kernel(x, edge_index, W1, a_src1, a_dst1, b1, W2, a_src2, a_dst2, b2):
    n = x.shape[0]
    e = edge_index.shape[1]
    heads, hid = a_src1.shape
    feat1 = heads * hid
    out_dim = W2.shape[1]
    dt1, dt2 = feat1 + 16, out_dim + 16

    workers = 32
    chunk = 40
    n_chunks = e // (workers * chunk)

    src_w = edge_index[0].reshape(workers, n_chunks, chunk)
    dst_w = edge_index[1].reshape(workers, n_chunks, chunk)

    f32 = jnp.float32
    z8 = jnp.zeros((x.shape[1], 16 - heads), f32)
    wa1 = jnp.concatenate([W1, W1 @ _head_matrix(a_src1), z8], axis=1)
    wd1 = jnp.concatenate([W1 @ _head_matrix(a_dst1), z8], axis=1)

    t1, d1 = _proj(x, wa1, wd1)
    acc1 = _make_sc_gat(n, dt1, feat1, hid, n_chunks, chunk)(
        t1, d1, src_w, dst_w)

    sel1 = jnp.concatenate(
        [jnp.zeros((feat1, feat1), f32),
         jnp.repeat(jnp.eye(heads, dtype=f32), hid, axis=1),
         jnp.zeros((16 - heads, feat1), f32)], axis=0)

    z15 = jnp.zeros((feat1, 15), f32)
    wa2 = jnp.concatenate([W2, W2 @ a_src2.T, z15], axis=1)
    wd2 = jnp.concatenate([W2 @ a_dst2.T, z15], axis=1)

    t2, d2 = _norm_proj(acc1[0], acc1[1], sel1, b1.reshape(1, feat1),
                        wa2, wd2)
    acc2 = _make_sc_gat(n, dt2, out_dim, out_dim, n_chunks, chunk)(
        t2, d2, src_w, dst_w)

    sel2 = jnp.concatenate(
        [jnp.zeros((out_dim, out_dim), f32),
         jnp.ones((1, out_dim), f32),
         jnp.zeros((15, out_dim), f32)], axis=0)

    return _final(acc2[0], acc2[1], sel2, b2.reshape(1, out_dim))
